# baseline (speedup 1.0000x reference)
import functools

import jax
import jax.numpy as jnp
import numpy as np
from jax import lax
from jax.experimental import pallas as pl
from jax.experimental.pallas import tpu as pltpu
from jax.experimental.pallas import tpu_sc as plsc

N = 10000
E = 320000
C = 128
H = 8
D = 16
NRBF = 50
CUT_UPPER = 5.0

NC = 2
NS = 16
NW = NC * NS

_j = np.arange(3 * C)
_t, _r = _j // C, _j % C
_h, _d = _r // D, _r % D
_PERM = (_h * (3 * D) + _t * D + _d).astype(np.int32)

_SUM_H = np.repeat(np.eye(H, dtype=np.float32), D, axis=0)
_EXP_H = _SUM_H.T.copy()
_EXP3 = np.repeat(np.eye(3, dtype=np.float32), C, axis=1)


def _silu(x):
    return x * (1.0 / (1.0 + jnp.exp(-x)))


def _pack2(a, b):
    ua = lax.bitcast_convert_type(
        a.astype(jnp.bfloat16).astype(jnp.float32), jnp.uint32)
    ub = lax.bitcast_convert_type(
        b.astype(jnp.bfloat16).astype(jnp.float32), jnp.uint32)
    return lax.bitcast_convert_type(ua | (ub >> 16), jnp.float32)


def _unpack_hi(p):
    u = lax.bitcast_convert_type(p, jnp.uint32)
    return lax.bitcast_convert_type(u & jnp.uint32(0xFFFF0000), jnp.float32)


def _unpack_lo(p):
    u = lax.bitcast_convert_type(p, jnp.uint32)
    return lax.bitcast_convert_type(u << 16, jnp.float32)


def _p1_body(x_ref, vec_ref, lng_ref, lnb_ref, wq_ref, bq_ref, wk_ref,
             bk_ref, wv_ref, bv_ref, wvec_ref,
             q_ref, t_ref, vdot_ref, vec3_ref):
    x = x_ref[...]
    m = jnp.mean(x, axis=-1, keepdims=True)
    v = jnp.mean((x - m) ** 2, axis=-1, keepdims=True)
    xn = (x - m) / jnp.sqrt(v + 1e-5) * lng_ref[...] + lnb_ref[...]
    q_ref[...] = jnp.dot(xn, wq_ref[...], preferred_element_type=jnp.float32) + bq_ref[...]
    kk = jnp.dot(xn, wk_ref[...], preferred_element_type=jnp.float32) + bk_ref[...]
    vv = jnp.dot(xn, wv_ref[...], preferred_element_type=jnp.float32) + bv_ref[...]
    vdot = jnp.zeros_like(x)
    vecs = []
    for k in range(3):
        veck = vec_ref[:, k, :]
        vecs.append(veck)
        vp = jnp.dot(veck, wvec_ref[...], preferred_element_type=jnp.float32)
        vdot = vdot + vp[:, :C] * vp[:, C:2 * C]
        vec3_ref[:, k, :] = vp[:, 2 * C:]
    vdot_ref[...] = vdot
    t_ref[:, :C] = _pack2(kk, vecs[0])
    t_ref[:, C:2 * C] = _pack2(vv[:, :C], vecs[1])
    t_ref[:, 2 * C:3 * C] = _pack2(vv[:, C:2 * C], vecs[2])
    t_ref[:, 3 * C:4 * C] = _pack2(vv[:, 2 * C:], jnp.zeros_like(x))


def _p1_call(x, vec, ln_g, ln_b, Wq, bq, Wk, bk, Wv_p, bv_p, Wvec):
    B = 2000
    grid = N // B
    row = lambda i: (i, 0)
    row3 = lambda i: (i, 0, 0)
    full = lambda i: (0, 0)
    return pl.pallas_call(
        _p1_body,
        grid=(grid,),
        in_specs=[
            pl.BlockSpec((B, C), row),
            pl.BlockSpec((B, 3, C), row3),
            pl.BlockSpec((1, C), full), pl.BlockSpec((1, C), full),
            pl.BlockSpec((C, C), full), pl.BlockSpec((1, C), full),
            pl.BlockSpec((C, C), full), pl.BlockSpec((1, C), full),
            pl.BlockSpec((C, 3 * C), full), pl.BlockSpec((1, 3 * C), full),
            pl.BlockSpec((C, 3 * C), full),
        ],
        out_specs=[
            pl.BlockSpec((B, C), row),
            pl.BlockSpec((B, 4 * C), row),
            pl.BlockSpec((B, C), row),
            pl.BlockSpec((B, 3, C), row3),
        ],
        out_shape=[
            jax.ShapeDtypeStruct((N, C), jnp.float32),
            jax.ShapeDtypeStruct((N, 4 * C), jnp.float32),
            jax.ShapeDtypeStruct((N, C), jnp.float32),
            jax.ShapeDtypeStruct((N, 3, C), jnp.float32),
        ],
    )(x, vec, ln_g.reshape(1, C), ln_b.reshape(1, C), Wq, bq.reshape(1, C),
      Wk, bk.reshape(1, C), Wv_p, bv_p.reshape(1, 3 * C), Wvec)


_EH = E // 2
_GB = 40
_EPW = _EH // NW
_GNB = _EPW // _GB


def _gather_body(t_hbm, q_hbm, src_hbm, dst_hbm, tj_out, qi_out,
                 idxs_v, idxd_v, tb, qb, semg, semw):
    wid = lax.axis_index("s") * NC + lax.axis_index("c")
    wbase = pl.multiple_of(wid * _EPW, 8)
    pltpu.sync_copy(src_hbm.at[pl.ds(wbase, _EPW)], idxs_v)
    pltpu.sync_copy(dst_hbm.at[pl.ds(wbase, _EPW)], idxd_v)

    def g_issue(j, b):
        sl = pl.ds(j * _GB, _GB)
        pltpu.async_copy(t_hbm.at[idxs_v.at[sl]], tb.at[b], semg.at[b])
        pltpu.async_copy(q_hbm.at[idxd_v.at[sl]], qb.at[b], semg.at[b])

    def g_wait(b):
        sl = pl.ds(0, _GB)
        pltpu.make_async_copy(t_hbm.at[idxs_v.at[sl]], tb.at[b], semg.at[b]).wait()
        pltpu.make_async_copy(q_hbm.at[idxd_v.at[sl]], qb.at[b], semg.at[b]).wait()

    def w_issue(j, b):
        base = pl.multiple_of(wbase + j * _GB, 8)
        pltpu.async_copy(tb.at[b], tj_out.at[pl.ds(base, _GB)], semw.at[b])
        pltpu.async_copy(qb.at[b], qi_out.at[pl.ds(base, _GB)], semw.at[b])

    def w_wait(b):
        base = pl.multiple_of(wbase, 8)
        pltpu.make_async_copy(tb.at[b], tj_out.at[pl.ds(base, _GB)], semw.at[b]).wait()
        pltpu.make_async_copy(qb.at[b], qi_out.at[pl.ds(base, _GB)], semw.at[b]).wait()

    g_issue(0, 0)
    g_issue(1, 1)
    g_wait(0)
    w_issue(0, 0)

    def pair(t, _):
        j = 2 * t
        w_wait(0)
        g_issue(j, 0)
        g_wait(1)
        w_issue(j - 1, 1)
        w_wait(1)
        g_issue(j + 1, 1)
        g_wait(0)
        w_issue(j, 0)
        return _

    lax.fori_loop(1, _GNB // 2, pair, None)
    w_wait(0)
    g_issue(_GNB - 1, 0)
    g_wait(1)
    w_issue(_GNB - 2, 1)
    g_wait(0)
    w_issue(_GNB - 1, 0)
    w_wait(1)
    w_wait(0)


def _gather_call(T, q, src, dst):
    mesh = plsc.VectorSubcoreMesh(core_axis_name="c", subcore_axis_name="s",
                                  num_cores=NC, num_subcores=NS)
    f = pl.kernel(
        _gather_body,
        out_type=[
            jax.ShapeDtypeStruct((_EH, 4 * C), jnp.float32),
            jax.ShapeDtypeStruct((_EH, C), jnp.float32),
        ],
        mesh=mesh,
        scratch_types=[
            pltpu.VMEM((_EPW,), jnp.int32),
            pltpu.VMEM((_EPW,), jnp.int32),
            pltpu.VMEM((2, _GB, 4 * C), jnp.float32),
            pltpu.VMEM((2, _GB, C), jnp.float32),
            pltpu.SemaphoreType.DMA((2,)),
            pltpu.SemaphoreType.DMA((2,)),
        ],
    )
    return f(T, q, src, dst)


def _p0_body(r_ref, cut_ref):
    r = r_ref[...]
    cut = 0.5 * (jnp.cos(r * (np.pi / CUT_UPPER)) + 1.0)
    cut_ref[...] = jnp.where(r < CUT_UPPER, cut, 0.0)


def _p0_call(r_ij):
    r2 = r_ij.reshape(E // C, C)
    out = pl.pallas_call(
        _p0_body,
        out_shape=jax.ShapeDtypeStruct((E // C, C), jnp.float32),
    )(r2)
    return out.reshape(E, 1)


def _p2_body(tj_ref, qi_ref, f_ref, r_ref, dij_ref, wdk_ref, bdk_ref,
             wdv_ref, bdv_ref, sumh_ref, exph_ref, exp3_ref, p_ref):
    fb = f_ref[...]
    dk = _silu(jnp.dot(fb, wdk_ref[...], preferred_element_type=jnp.float32) + bdk_ref[...])
    dv = _silu(jnp.dot(fb, wdv_ref[...], preferred_element_type=jnp.float32) + bdv_ref[...])
    tj = tj_ref[...]
    hi = _unpack_hi(tj)
    lo = _unpack_lo(tj)
    prod = qi_ref[...] * hi[:, :C] * dk[:, :C]
    attn = jnp.dot(prod, sumh_ref[...], preferred_element_type=jnp.float32)
    attn = _silu(attn) * r_ref[...]
    attn128 = jnp.dot(attn, exph_ref[...], preferred_element_type=jnp.float32)
    xm = hi[:, C:2 * C] * dv[:, :C] * attn128
    v1m = hi[:, 2 * C:3 * C] * dv[:, C:2 * C]
    v2m = hi[:, 3 * C:4 * C] * dv[:, 2 * C:3 * C]
    dbc = jnp.dot(dij_ref[...], exp3_ref[...], preferred_element_type=jnp.float32)
    p_ref[:, :C] = xm
    for k in range(3):
        p_ref[:, (k + 1) * C:(k + 2) * C] = (
            lo[:, k * C:(k + 1) * C] * v1m + v2m * dbc[:, k * C:(k + 1) * C])


def _p2_call(TJ, QI, f_ij, r_col, d_ij, Wdk, bdk, Wdv_p, bdv_p):
    EB = 1280
    grid = _EH // EB
    row = lambda i: (i, 0)
    full = lambda i: (0, 0)
    return pl.pallas_call(
        _p2_body,
        grid=(grid,),
        in_specs=[
            pl.BlockSpec((EB, 4 * C), row),
            pl.BlockSpec((EB, C), row),
            pl.BlockSpec((EB, NRBF), row),
            pl.BlockSpec((EB, 1), row),
            pl.BlockSpec((EB, 3), row),
            pl.BlockSpec((NRBF, C), full), pl.BlockSpec((1, C), full),
            pl.BlockSpec((NRBF, 3 * C), full), pl.BlockSpec((1, 3 * C), full),
            pl.BlockSpec((C, H), full), pl.BlockSpec((H, C), full),
            pl.BlockSpec((3, 3 * C), full),
        ],
        out_specs=pl.BlockSpec((EB, 4 * C), row),
        out_shape=jax.ShapeDtypeStruct((_EH, 4 * C), jnp.float32),
    )(TJ, QI, f_ij, r_col, d_ij, Wdk, bdk.reshape(1, C), Wdv_p,
      bdv_p.reshape(1, 3 * C), jnp.asarray(_SUM_H), jnp.asarray(_EXP_H),
      jnp.asarray(_EXP3))


_SB = 40
_EPT = E // NS
_ZR = 1000


_SBK = 128
_SROWS = E // _SBK
_SPAD = 2504
_RPT = 160
_NG = _RPT // 8


_SROWS_H = _EH // _SBK


def _scatter_body(pa_hbm, pb_hbm, dst2_hbm, zeros_hbm, agg_out, idxw, pbuf,
                  acc, seml, sems):
    cid = lax.axis_index("c")
    sid = lax.axis_index("s")
    r0 = pl.multiple_of(sid * _ZR, 8)
    ra = sid * _RPT

    def ok(jj):
        return jnp.logical_and(jj >= 0, ra + jj < _SROWS)

    for p in range(2):
        fc = pl.multiple_of((cid * 2 + p) * C, 128)
        @pl.when(sid < N // _ZR)
        def _zero():
            pltpu.sync_copy(zeros_hbm.at[pl.ds(0, _ZR)], acc.at[pl.ds(r0, _ZR)])
        plsc.subcore_barrier()

        def l_issue(j, b):
            r = ra + j
            @pl.when(r < _SROWS_H)
            def _a():
                row = pl.multiple_of(r * _SBK, 8)
                pltpu.async_copy(pa_hbm.at[pl.ds(row, _SBK), pl.ds(fc, C)],
                                 pbuf.at[b], seml.at[b])
            @pl.when(r >= _SROWS_H)
            def _b():
                row = pl.multiple_of((r - _SROWS_H) * _SBK, 8)
                pltpu.async_copy(pb_hbm.at[pl.ds(row, _SBK), pl.ds(fc, C)],
                                 pbuf.at[b], seml.at[b])

        def l_wait(b):
            pltpu.make_async_copy(pa_hbm.at[pl.ds(0, _SBK), pl.ds(fc, C)],
                                  pbuf.at[b], seml.at[b]).wait()

        def s_issue(b, gp, k):
            pltpu.async_copy(pbuf.at[b], acc.at[idxw.at[gp, k]],
                             sems.at[b], add=True)

        def s_wait(b):
            pltpu.make_async_copy(pbuf.at[b], acc.at[idxw.at[0, 0]],
                                  sems.at[b]).wait()

        def group(g, _):
            gp = lax.rem(g, 2)
            gprev = lax.rem(g + 1, 2)
            @pl.when(ok(8 * g))
            def _ld_idx():
                base = pl.multiple_of(ra + 8 * g, 8)
                pltpu.sync_copy(dst2_hbm.at[pl.ds(base, 8)], idxw.at[gp])
            for k in range(8):
                j = 8 * g + k
                @pl.when(ok(j - 2))
                def _sw():
                    s_wait(k % 2)
                @pl.when(ok(j))
                def _l():
                    l_issue(j, k % 2)
                @pl.when(ok(j - 1))
                def _lw_s():
                    l_wait((k + 1) % 2)
                    s_issue((k + 1) % 2, gp if k >= 1 else gprev, (k - 1) % 8)
            return _

        lax.fori_loop(0, _NG, group, None, unroll=False)
        jlast = _RPT - 1
        @pl.when(ok(jlast))
        def _tail1():
            l_wait(jlast % 2)
            s_issue(jlast % 2, lax.rem(_NG - 1, 2), 7)
        @pl.when(ok(jlast - 1))
        def _tail2():
            s_wait((jlast - 1) % 2)
        @pl.when(ok(jlast))
        def _tail3():
            s_wait(jlast % 2)
        plsc.subcore_barrier()

        @pl.when(sid < N // _ZR)
        def _writeback():
            pltpu.sync_copy(acc.at[pl.ds(r0, _ZR)],
                            agg_out.at[pl.ds(r0, _ZR), pl.ds(fc, C)])
        plsc.subcore_barrier()


def _scatter_call(Pa, Pb, dst2, zeros_rows):
    mesh = plsc.VectorSubcoreMesh(core_axis_name="c", subcore_axis_name="s",
                                  num_cores=NC, num_subcores=NS)
    f = pl.kernel(
        _scatter_body,
        out_type=jax.ShapeDtypeStruct((N, 4 * C), jnp.float32),
        mesh=mesh,
        scratch_types=[
            pltpu.VMEM((2, 8, _SBK), jnp.int32),
            pltpu.VMEM((2, _SBK, C), jnp.float32),
            pltpu.VMEM_SHARED((N, C), jnp.float32),
            pltpu.SemaphoreType.DMA((2,)),
            pltpu.SemaphoreType.DMA((2,)),
        ],
    )
    return f(Pa, Pb, dst2, zeros_rows)


def _p3_body(agg_ref, vdot_ref, vec3_ref, wo_ref, bo_ref, dx_ref, dvec_ref):
    o = jnp.dot(agg_ref[:, :C], wo_ref[...], preferred_element_type=jnp.float32) + bo_ref[...]
    dx_ref[...] = vdot_ref[...] * o[:, C:2 * C] + o[:, 2 * C:]
    for k in range(3):
        dvec_ref[:, k, :] = vec3_ref[:, k, :] * o[:, :C] + agg_ref[:, (k + 1) * C:(k + 2) * C]


def _p3_call(AGG, vdot, vec3, Wo, bo):
    B = 1000
    grid = N // B
    row = lambda i: (i, 0)
    row3 = lambda i: (i, 0, 0)
    full = lambda i: (0, 0)
    return pl.pallas_call(
        _p3_body,
        grid=(grid,),
        in_specs=[
            pl.BlockSpec((B, 4 * C), row),
            pl.BlockSpec((B, C), row),
            pl.BlockSpec((B, 3, C), row3),
            pl.BlockSpec((C, 3 * C), full), pl.BlockSpec((1, 3 * C), full),
        ],
        out_specs=[
            pl.BlockSpec((B, C), row),
            pl.BlockSpec((B, 3, C), row3),
        ],
        out_shape=[
            jax.ShapeDtypeStruct((N, C), jnp.float32),
            jax.ShapeDtypeStruct((N, 3, C), jnp.float32),
        ],
    )(AGG, vdot, vec3, Wo, bo.reshape(1, 3 * C))


def kernel(x, vec, edge_index, r_ij, f_ij, d_ij, ln_g, ln_b, Wq, bq, Wk, bk,
           Wv, bv, Wvec, Wo, bo, Wdk, bdk, Wdv, bdv):
    perm = jnp.asarray(_PERM)
    Wv_p, bv_p = Wv[:, perm], bv[perm]
    Wdv_p, bdv_p = Wdv[:, perm], bdv[perm]

    q, T, vdot, vec3 = _p1_call(x, vec, ln_g, ln_b, Wq, bq, Wk, bk,
                                Wv_p, bv_p, Wvec)
    src = edge_index[0]
    dst = edge_index[1]
    cut = _p0_call(r_ij)
    halves = []
    for h in range(2):
        sl = slice(h * _EH, (h + 1) * _EH)
        TJ, QI = _gather_call(T, q, src[sl], dst[sl])
        halves.append(_p2_call(TJ, QI, f_ij[sl], cut[sl], d_ij[sl],
                               Wdk, bdk, Wdv_p, bdv_p))
    zeros_rows = jnp.zeros((_ZR, C), jnp.float32)
    dst2 = jnp.concatenate(
        [dst, jnp.zeros((_SPAD * _SBK - E,), jnp.int32)]).reshape(_SPAD, _SBK)
    AGG = _scatter_call(halves[0], halves[1], dst2, zeros_rows)
    dx, dvec = _p3_call(AGG, vdot, vec3, Wo, bo)
    return (dx, dvec)


# speedup vs baseline: 1.0643x; 1.0643x over previous
import functools

import jax
import jax.numpy as jnp
import numpy as np
from jax import lax
from jax.experimental import pallas as pl
from jax.experimental.pallas import tpu as pltpu
from jax.experimental.pallas import tpu_sc as plsc

N = 10000
E = 320000
C = 128
H = 8
D = 16
NRBF = 50
CUT_UPPER = 5.0

NC = 2
NS = 16
NW = NC * NS

_j = np.arange(3 * C)
_t, _r = _j // C, _j % C
_h, _d = _r // D, _r % D
_PERM = (_h * (3 * D) + _t * D + _d).astype(np.int32)

_SUM_H = np.repeat(np.eye(H, dtype=np.float32), D, axis=0)
_EXP_H = _SUM_H.T.copy()
_EXP3 = np.repeat(np.eye(3, dtype=np.float32), C, axis=1)


def _silu(x):
    return x * (1.0 / (1.0 + jnp.exp(-x)))


def _pack2(a, b):
    ua = lax.bitcast_convert_type(
        a.astype(jnp.bfloat16).astype(jnp.float32), jnp.uint32)
    ub = lax.bitcast_convert_type(
        b.astype(jnp.bfloat16).astype(jnp.float32), jnp.uint32)
    return lax.bitcast_convert_type(ua | (ub >> 16), jnp.float32)


def _unpack_hi(p):
    u = lax.bitcast_convert_type(p, jnp.uint32)
    return lax.bitcast_convert_type(u & jnp.uint32(0xFFFF0000), jnp.float32)


def _unpack_lo(p):
    u = lax.bitcast_convert_type(p, jnp.uint32)
    return lax.bitcast_convert_type(u << 16, jnp.float32)


def _p1_body(x_ref, vec_ref, lng_ref, lnb_ref, wq_ref, bq_ref, wk_ref,
             bk_ref, wv_ref, bv_ref, wvec_ref,
             q_ref, t_ref, vdot_ref, vec3_ref):
    x = x_ref[...]
    m = jnp.mean(x, axis=-1, keepdims=True)
    v = jnp.mean((x - m) ** 2, axis=-1, keepdims=True)
    xn = (x - m) / jnp.sqrt(v + 1e-5) * lng_ref[...] + lnb_ref[...]
    q_ref[...] = jnp.dot(xn, wq_ref[...], preferred_element_type=jnp.float32) + bq_ref[...]
    kk = jnp.dot(xn, wk_ref[...], preferred_element_type=jnp.float32) + bk_ref[...]
    vv = jnp.dot(xn, wv_ref[...], preferred_element_type=jnp.float32) + bv_ref[...]
    vdot = jnp.zeros_like(x)
    vecs = []
    for k in range(3):
        veck = vec_ref[:, k, :]
        vecs.append(veck)
        vp = jnp.dot(veck, wvec_ref[...], preferred_element_type=jnp.float32)
        vdot = vdot + vp[:, :C] * vp[:, C:2 * C]
        vec3_ref[:, k, :] = vp[:, 2 * C:]
    vdot_ref[...] = vdot
    t_ref[:, :C] = _pack2(kk, vecs[0])
    t_ref[:, C:2 * C] = _pack2(vv[:, :C], vecs[1])
    t_ref[:, 2 * C:3 * C] = _pack2(vv[:, C:2 * C], vecs[2])
    t_ref[:, 3 * C:4 * C] = _pack2(vv[:, 2 * C:], jnp.zeros_like(x))


def _p1_call(x, vec, ln_g, ln_b, Wq, bq, Wk, bk, Wv_p, bv_p, Wvec):
    B = 2000
    grid = N // B
    row = lambda i: (i, 0)
    row3 = lambda i: (i, 0, 0)
    full = lambda i: (0, 0)
    return pl.pallas_call(
        _p1_body,
        grid=(grid,),
        in_specs=[
            pl.BlockSpec((B, C), row),
            pl.BlockSpec((B, 3, C), row3),
            pl.BlockSpec((1, C), full), pl.BlockSpec((1, C), full),
            pl.BlockSpec((C, C), full), pl.BlockSpec((1, C), full),
            pl.BlockSpec((C, C), full), pl.BlockSpec((1, C), full),
            pl.BlockSpec((C, 3 * C), full), pl.BlockSpec((1, 3 * C), full),
            pl.BlockSpec((C, 3 * C), full),
        ],
        out_specs=[
            pl.BlockSpec((B, C), row),
            pl.BlockSpec((B, 4 * C), row),
            pl.BlockSpec((B, C), row),
            pl.BlockSpec((B, 3, C), row3),
        ],
        out_shape=[
            jax.ShapeDtypeStruct((N, C), jnp.float32),
            jax.ShapeDtypeStruct((N, 4 * C), jnp.float32),
            jax.ShapeDtypeStruct((N, C), jnp.float32),
            jax.ShapeDtypeStruct((N, 3, C), jnp.float32),
        ],
    )(x, vec, ln_g.reshape(1, C), ln_b.reshape(1, C), Wq, bq.reshape(1, C),
      Wk, bk.reshape(1, C), Wv_p, bv_p.reshape(1, 3 * C), Wvec)


_EH = E
_GB = 80
_EPW = _EH // NW
_GNB = _EPW // _GB


def _gather_body(t_hbm, q_hbm, src_hbm, dst_hbm, tj_out, qi_out,
                 idxs_v, idxd_v, tb, qb, semg, semw):
    wid = lax.axis_index("s") * NC + lax.axis_index("c")
    wbase = pl.multiple_of(wid * _EPW, 8)
    pltpu.sync_copy(src_hbm.at[pl.ds(wbase, _EPW)], idxs_v)
    pltpu.sync_copy(dst_hbm.at[pl.ds(wbase, _EPW)], idxd_v)

    def g_issue(j, b):
        sl = pl.ds(j * _GB, _GB)
        pltpu.async_copy(t_hbm.at[idxs_v.at[sl]], tb.at[b], semg.at[b])
        pltpu.async_copy(q_hbm.at[idxd_v.at[sl]], qb.at[b], semg.at[b])

    def g_wait(b):
        sl = pl.ds(0, _GB)
        pltpu.make_async_copy(t_hbm.at[idxs_v.at[sl]], tb.at[b], semg.at[b]).wait()
        pltpu.make_async_copy(q_hbm.at[idxd_v.at[sl]], qb.at[b], semg.at[b]).wait()

    def w_issue(j, b):
        base = pl.multiple_of(wbase + j * _GB, 8)
        pltpu.async_copy(tb.at[b], tj_out.at[pl.ds(base, _GB)], semw.at[b])
        pltpu.async_copy(qb.at[b], qi_out.at[pl.ds(base, _GB)], semw.at[b])

    def w_wait(b):
        base = pl.multiple_of(wbase, 8)
        pltpu.make_async_copy(tb.at[b], tj_out.at[pl.ds(base, _GB)], semw.at[b]).wait()
        pltpu.make_async_copy(qb.at[b], qi_out.at[pl.ds(base, _GB)], semw.at[b]).wait()

    g_issue(0, 0)
    g_issue(1, 1)
    g_wait(0)
    w_issue(0, 0)

    def pair(t, _):
        j = 2 * t
        w_wait(0)
        g_issue(j, 0)
        g_wait(1)
        w_issue(j - 1, 1)
        w_wait(1)
        g_issue(j + 1, 1)
        g_wait(0)
        w_issue(j, 0)
        return _

    lax.fori_loop(1, _GNB // 2, pair, None)
    w_wait(0)
    g_issue(_GNB - 1, 0)
    g_wait(1)
    w_issue(_GNB - 2, 1)
    g_wait(0)
    w_issue(_GNB - 1, 0)
    w_wait(1)
    w_wait(0)


def _gather_call(T, q, src, dst):
    mesh = plsc.VectorSubcoreMesh(core_axis_name="c", subcore_axis_name="s",
                                  num_cores=NC, num_subcores=NS)
    f = pl.kernel(
        _gather_body,
        out_type=[
            jax.ShapeDtypeStruct((_EH, 4 * C), jnp.float32),
            jax.ShapeDtypeStruct((_EH, C), jnp.float32),
        ],
        mesh=mesh,
        scratch_types=[
            pltpu.VMEM((_EPW,), jnp.int32),
            pltpu.VMEM((_EPW,), jnp.int32),
            pltpu.VMEM((2, _GB, 4 * C), jnp.float32),
            pltpu.VMEM((2, _GB, C), jnp.float32),
            pltpu.SemaphoreType.DMA((2,)),
            pltpu.SemaphoreType.DMA((2,)),
        ],
    )
    return f(T, q, src, dst)


def _p0_body(r_ref, cut_ref):
    r = r_ref[...]
    cut = 0.5 * (jnp.cos(r * (np.pi / CUT_UPPER)) + 1.0)
    cut_ref[...] = jnp.where(r < CUT_UPPER, cut, 0.0)


def _p0_call(r_ij):
    r2 = r_ij.reshape(E // C, C)
    out = pl.pallas_call(
        _p0_body,
        out_shape=jax.ShapeDtypeStruct((E // C, C), jnp.float32),
    )(r2)
    return out.reshape(E, 1)


def _p2_body(tj_ref, qi_ref, f_ref, r_ref, dij_ref, wdk_ref, bdk_ref,
             wdv_ref, bdv_ref, sumh_ref, exph_ref, exp3_ref, p_ref):
    fb = f_ref[...]
    dk = _silu(jnp.dot(fb, wdk_ref[...], preferred_element_type=jnp.float32) + bdk_ref[...])
    dv = _silu(jnp.dot(fb, wdv_ref[...], preferred_element_type=jnp.float32) + bdv_ref[...])
    tj = tj_ref[...]
    hi = _unpack_hi(tj)
    lo = _unpack_lo(tj)
    prod = qi_ref[...] * hi[:, :C] * dk[:, :C]
    attn = jnp.dot(prod, sumh_ref[...], preferred_element_type=jnp.float32)
    attn = _silu(attn) * r_ref[...]
    attn128 = jnp.dot(attn, exph_ref[...], preferred_element_type=jnp.float32)
    xm = hi[:, C:2 * C] * dv[:, :C] * attn128
    v1m = hi[:, 2 * C:3 * C] * dv[:, C:2 * C]
    v2m = hi[:, 3 * C:4 * C] * dv[:, 2 * C:3 * C]
    dbc = jnp.dot(dij_ref[...], exp3_ref[...], preferred_element_type=jnp.float32)
    p_ref[:, :C] = xm
    for k in range(3):
        p_ref[:, (k + 1) * C:(k + 2) * C] = (
            lo[:, k * C:(k + 1) * C] * v1m + v2m * dbc[:, k * C:(k + 1) * C])


def _p2_call(TJ, QI, f_ij, r_col, d_ij, Wdk, bdk, Wdv_p, bdv_p):
    EB = 1280
    grid = _EH // EB
    row = lambda i: (i, 0)
    full = lambda i: (0, 0)
    return pl.pallas_call(
        _p2_body,
        grid=(grid,),
        in_specs=[
            pl.BlockSpec((EB, 4 * C), row),
            pl.BlockSpec((EB, C), row),
            pl.BlockSpec((EB, NRBF), row),
            pl.BlockSpec((EB, 1), row),
            pl.BlockSpec((EB, 3), row),
            pl.BlockSpec((NRBF, C), full), pl.BlockSpec((1, C), full),
            pl.BlockSpec((NRBF, 3 * C), full), pl.BlockSpec((1, 3 * C), full),
            pl.BlockSpec((C, H), full), pl.BlockSpec((H, C), full),
            pl.BlockSpec((3, 3 * C), full),
        ],
        out_specs=pl.BlockSpec((EB, 4 * C), row),
        out_shape=jax.ShapeDtypeStruct((_EH, 4 * C), jnp.float32),
    )(TJ, QI, f_ij, r_col, d_ij, Wdk, bdk.reshape(1, C), Wdv_p,
      bdv_p.reshape(1, 3 * C), jnp.asarray(_SUM_H), jnp.asarray(_EXP_H),
      jnp.asarray(_EXP3))


_SB = 40
_EPT = E // NS
_ZR = 1000


_SBK = 128
_SROWS = E // _SBK
_SPAD = 2504
_RPT = 160
_NG = _RPT // 8


_SROWS_H = _EH // _SBK


def _scatter_body(pa_hbm, pb_hbm, dst2_hbm, zeros_hbm, agg_out, idxw, pbuf,
                  acc, seml, sems):
    cid = lax.axis_index("c")
    sid = lax.axis_index("s")
    r0 = pl.multiple_of(sid * _ZR, 8)
    ra = sid * _RPT

    def ok(jj):
        return jnp.logical_and(jj >= 0, ra + jj < _SROWS)

    for p in range(2):
        fc = pl.multiple_of((cid * 2 + p) * C, 128)
        @pl.when(sid < N // _ZR)
        def _zero():
            pltpu.sync_copy(zeros_hbm.at[pl.ds(0, _ZR)], acc.at[pl.ds(r0, _ZR)])
        plsc.subcore_barrier()

        def l_issue(j, b):
            r = ra + j
            @pl.when(r < _SROWS_H)
            def _a():
                row = pl.multiple_of(r * _SBK, 8)
                pltpu.async_copy(pa_hbm.at[pl.ds(row, _SBK), pl.ds(fc, C)],
                                 pbuf.at[b], seml.at[b])
            @pl.when(r >= _SROWS_H)
            def _b():
                row = pl.multiple_of((r - _SROWS_H) * _SBK, 8)
                pltpu.async_copy(pb_hbm.at[pl.ds(row, _SBK), pl.ds(fc, C)],
                                 pbuf.at[b], seml.at[b])

        def l_wait(b):
            pltpu.make_async_copy(pa_hbm.at[pl.ds(0, _SBK), pl.ds(fc, C)],
                                  pbuf.at[b], seml.at[b]).wait()

        def s_issue(b, gp, k):
            pltpu.async_copy(pbuf.at[b], acc.at[idxw.at[gp, k]],
                             sems.at[b], add=True)

        def s_wait(b):
            pltpu.make_async_copy(pbuf.at[b], acc.at[idxw.at[0, 0]],
                                  sems.at[b]).wait()

        def group(g, _):
            gp = lax.rem(g, 2)
            gprev = lax.rem(g + 1, 2)
            @pl.when(ok(8 * g))
            def _ld_idx():
                base = pl.multiple_of(ra + 8 * g, 8)
                pltpu.sync_copy(dst2_hbm.at[pl.ds(base, 8)], idxw.at[gp])
            for k in range(8):
                j = 8 * g + k
                @pl.when(ok(j - 2))
                def _sw():
                    s_wait(k % 2)
                @pl.when(ok(j))
                def _l():
                    l_issue(j, k % 2)
                @pl.when(ok(j - 1))
                def _lw_s():
                    l_wait((k + 1) % 2)
                    s_issue((k + 1) % 2, gp if k >= 1 else gprev, (k - 1) % 8)
            return _

        lax.fori_loop(0, _NG, group, None, unroll=False)
        jlast = _RPT - 1
        @pl.when(ok(jlast))
        def _tail1():
            l_wait(jlast % 2)
            s_issue(jlast % 2, lax.rem(_NG - 1, 2), 7)
        @pl.when(ok(jlast - 1))
        def _tail2():
            s_wait((jlast - 1) % 2)
        @pl.when(ok(jlast))
        def _tail3():
            s_wait(jlast % 2)
        plsc.subcore_barrier()

        @pl.when(sid < N // _ZR)
        def _writeback():
            pltpu.sync_copy(acc.at[pl.ds(r0, _ZR)],
                            agg_out.at[pl.ds(r0, _ZR), pl.ds(fc, C)])
        plsc.subcore_barrier()


def _scatter_call(Pa, Pb, dst2, zeros_rows):
    mesh = plsc.VectorSubcoreMesh(core_axis_name="c", subcore_axis_name="s",
                                  num_cores=NC, num_subcores=NS)
    f = pl.kernel(
        _scatter_body,
        out_type=jax.ShapeDtypeStruct((N, 4 * C), jnp.float32),
        mesh=mesh,
        scratch_types=[
            pltpu.VMEM((2, 8, _SBK), jnp.int32),
            pltpu.VMEM((2, _SBK, C), jnp.float32),
            pltpu.VMEM_SHARED((N, C), jnp.float32),
            pltpu.SemaphoreType.DMA((2,)),
            pltpu.SemaphoreType.DMA((2,)),
        ],
    )
    return f(Pa, Pb, dst2, zeros_rows)


def _p3_body(agg_ref, vdot_ref, vec3_ref, wo_ref, bo_ref, dx_ref, dvec_ref):
    o = jnp.dot(agg_ref[:, :C], wo_ref[...], preferred_element_type=jnp.float32) + bo_ref[...]
    dx_ref[...] = vdot_ref[...] * o[:, C:2 * C] + o[:, 2 * C:]
    for k in range(3):
        dvec_ref[:, k, :] = vec3_ref[:, k, :] * o[:, :C] + agg_ref[:, (k + 1) * C:(k + 2) * C]


def _p3_call(AGG, vdot, vec3, Wo, bo):
    B = 1000
    grid = N // B
    row = lambda i: (i, 0)
    row3 = lambda i: (i, 0, 0)
    full = lambda i: (0, 0)
    return pl.pallas_call(
        _p3_body,
        grid=(grid,),
        in_specs=[
            pl.BlockSpec((B, 4 * C), row),
            pl.BlockSpec((B, C), row),
            pl.BlockSpec((B, 3, C), row3),
            pl.BlockSpec((C, 3 * C), full), pl.BlockSpec((1, 3 * C), full),
        ],
        out_specs=[
            pl.BlockSpec((B, C), row),
            pl.BlockSpec((B, 3, C), row3),
        ],
        out_shape=[
            jax.ShapeDtypeStruct((N, C), jnp.float32),
            jax.ShapeDtypeStruct((N, 3, C), jnp.float32),
        ],
    )(AGG, vdot, vec3, Wo, bo.reshape(1, 3 * C))


def kernel(x, vec, edge_index, r_ij, f_ij, d_ij, ln_g, ln_b, Wq, bq, Wk, bk,
           Wv, bv, Wvec, Wo, bo, Wdk, bdk, Wdv, bdv):
    perm = jnp.asarray(_PERM)
    Wv_p, bv_p = Wv[:, perm], bv[perm]
    Wdv_p, bdv_p = Wdv[:, perm], bdv[perm]

    q, T, vdot, vec3 = _p1_call(x, vec, ln_g, ln_b, Wq, bq, Wk, bk,
                                Wv_p, bv_p, Wvec)
    src = edge_index[0]
    dst = edge_index[1]
    cut = _p0_call(r_ij)
    halves = []
    for h in range(E // _EH):
        sl = slice(h * _EH, (h + 1) * _EH)
        TJ, QI = _gather_call(T, q, src[sl], dst[sl])
        halves.append(_p2_call(TJ, QI, f_ij[sl], cut[sl], d_ij[sl],
                               Wdk, bdk, Wdv_p, bdv_p))
    zeros_rows = jnp.zeros((_ZR, C), jnp.float32)
    dst2 = jnp.concatenate(
        [dst, jnp.zeros((_SPAD * _SBK - E,), jnp.int32)]).reshape(_SPAD, _SBK)
    AGG = _scatter_call(halves[0], halves[-1], dst2, zeros_rows)
    dx, dvec = _p3_call(AGG, vdot, vec3, Wo, bo)
    return (dx, dvec)
